# baseline (speedup 1.0000x reference)
import functools

import jax
import jax.numpy as jnp
from jax import lax
from jax.experimental import pallas as pl
from jax.experimental.pallas import tpu as pltpu
from jax.experimental.pallas import tpu_sc as plsc

E = 8
K = 2
T = 2048
D = 1024
FF = 2048

BT = 256
N = T * K
NB = (N + E * (BT - 1) + BT - 1) // BT
P = NB * BT
NBH = NB // 2
PH = NBH * BT

NC = 2
NS = 16
NW = NC * NS

ROWS_W = PH // NW
CH = 48
NCH = ROWS_W // CH

TOK_W = T // NW
FCH = 32
NFCH = TOK_W // FCH


@functools.cache
def _build_sc_kernels():
    mesh = plsc.VectorSubcoreMesh(core_axis_name="c", subcore_axis_name="s")

    @functools.partial(
        pl.kernel,
        out_type=jax.ShapeDtypeStruct((T, D), jnp.float32),
        mesh=mesh,
        scratch_types=(
            [pltpu.VMEM((NFCH, 2 * FCH), jnp.int32)]
            + [pltpu.VMEM((2 * FCH, D), jnp.float32)]
            + [pltpu.VMEM((FCH, D), jnp.float32)]
            + [pltpu.SemaphoreType.DMA]
        ),
    )
    def sc_finalize(y_hbm, sint_hbm, out_hbm, *refs):
        idx_v = refs[0]
        g_v, o_v, sem = refs[1], refs[2], refs[3]
        wid = lax.axis_index("s") * NC + lax.axis_index("c")
        base = wid * TOK_W
        pltpu.sync_copy(sint_hbm.at[wid], idx_v)
        for c in range(NFCH):
            pltpu.async_copy(y_hbm.at[idx_v.at[c]], g_v, sem).wait()

            def _add(j, _):
                col = j * 16
                for r in range(FCH):
                    o_v[r, pl.ds(col, 16)] = (g_v[2 * r, pl.ds(col, 16)]
                                              + g_v[2 * r + 1, pl.ds(col, 16)])
                return 0

            lax.fori_loop(0, D // 16, _add, 0)
            pltpu.sync_copy(o_v, out_hbm.at[pl.ds(pl.multiple_of(base + c * FCH, 8), FCH)])

    return sc_finalize


def _tc_dispatch_body(nv_ref, hs_ref, tok_ref, x_ref):
    @pl.when(pl.program_id(0) < nv_ref[0])
    def _():
        tok = tok_ref[...]
        sel = (tok == lax.broadcasted_iota(jnp.int32, (BT, T), 1))
        s_mat = jnp.where(sel, 1.0, 0.0)
        x_ref[...] = lax.dot_general(
            s_mat, hs_ref[...], (((1,), (0,)), ((), ())),
            preferred_element_type=jnp.float32)


_tc_dispatch = pl.pallas_call(
    _tc_dispatch_body,
    grid_spec=pltpu.PrefetchScalarGridSpec(
        num_scalar_prefetch=1,
        grid=(NB,),
        in_specs=[
            pl.BlockSpec((T, D), lambda i, nv: (0, 0)),
            pl.BlockSpec((BT, 1), lambda i, nv: (i, 0)),
        ],
        out_specs=pl.BlockSpec((BT, D), lambda i, nv: (i, 0)),
    ),
    out_shape=jax.ShapeDtypeStruct((P, D), jnp.float32),
)


def _tc_moe_body(be_ref, nv_ref, x_ref, w1_ref, w2_ref, sw_ref, y_ref):
    @pl.when(pl.program_id(0) < nv_ref[0])
    def _():
        x = x_ref[...].astype(jnp.bfloat16)
        w1b = w1_ref[0].astype(jnp.bfloat16)
        g = lax.dot_general(x, w1b[:FF], (((1,), (1,)), ((), ())),
                            preferred_element_type=jnp.float32)
        u = lax.dot_general(x, w1b[FF:], (((1,), (1,)), ((), ())),
                            preferred_element_type=jnp.float32)
        h = (g * lax.logistic(g) * u).astype(jnp.bfloat16)
        y = lax.dot_general(h, w2_ref[0].astype(jnp.bfloat16),
                            (((1,), (1,)), ((), ())),
                            preferred_element_type=jnp.float32)
        y_ref[...] = y * sw_ref[...]


_tc_moe = pl.pallas_call(
    _tc_moe_body,
    grid_spec=pltpu.PrefetchScalarGridSpec(
        num_scalar_prefetch=2,
        grid=(NB,),
        in_specs=[
            pl.BlockSpec((BT, D), lambda i, be, nv: (i, 0)),
            pl.BlockSpec((1, 2 * FF, D), lambda i, be, nv: (be[i], 0, 0)),
            pl.BlockSpec((1, D, FF), lambda i, be, nv: (be[i], 0, 0)),
            pl.BlockSpec((BT, 1), lambda i, be, nv: (i, 0)),
        ],
        out_specs=pl.BlockSpec((BT, D), lambda i, be, nv: (i, 0)),
    ),
    out_shape=jax.ShapeDtypeStruct((P, D), jnp.float32),
)


def kernel(hidden_states, w1, w2, topk_weights, topk_ids):
    flat_e = topk_ids.reshape(-1).astype(jnp.int32)
    order = jnp.argsort(flat_e, stable=True)
    inv_order = jnp.argsort(order)
    sorted_e = flat_e[order]
    counts = jnp.bincount(flat_e, length=E)
    raw_off = jnp.cumsum(counts) - counts
    pad_counts = ((counts + BT - 1) // BT) * BT
    pad_cum = jnp.cumsum(pad_counts)
    pad_off = pad_cum - pad_counts
    n_valid = (jnp.sum(pad_counts) // BT).astype(jnp.int32)
    be_raw = jnp.minimum(
        jnp.searchsorted(pad_cum, jnp.arange(NB) * BT, side="right"), E - 1
    ).astype(jnp.int32)
    block_expert = jnp.where(jnp.arange(NB) < n_valid, be_raw,
                             be_raw[jnp.maximum(n_valid - 1, 0)]).astype(jnp.int32)
    slot = (pad_off[sorted_e] + jnp.arange(N) - raw_off[sorted_e]).astype(jnp.int32)
    packed = jnp.stack(
        [(order // K).astype(jnp.float32), topk_weights.reshape(-1)[order]])
    slot_info = jnp.zeros((2, P), jnp.float32).at[:, slot].set(packed)
    sorted_token = slot_info[0].astype(jnp.int32)
    sorted_wt = slot_info[1]
    sint = (pad_off[flat_e] + inv_order - raw_off[flat_e]).astype(
        jnp.int32).reshape(T, K)

    sc_finalize = _build_sc_kernels()
    nv = n_valid.reshape(1)
    x_sorted = _tc_dispatch(nv, hidden_states, sorted_token.reshape(P, 1))
    y_sorted = _tc_moe(block_expert, nv, x_sorted, w1, w2,
                       sorted_wt.reshape(P, 1))
    out = sc_finalize(y_sorted, sint.reshape(NW, NFCH, 2 * FCH))
    return out


# speedup vs baseline: 1.0050x; 1.0050x over previous
import functools

import jax
import jax.numpy as jnp
from jax import lax
from jax.experimental import pallas as pl
from jax.experimental.pallas import tpu as pltpu
from jax.experimental.pallas import tpu_sc as plsc

E = 8
K = 2
T = 2048
D = 1024
FF = 2048

BT = 256
N = T * K
NB = (N + E * (BT - 1) + BT - 1) // BT
P = NB * BT
NBH = NB // 2
PH = NBH * BT

NC = 2
NS = 16
NW = NC * NS

ROWS_W = PH // NW
CH = 48
NCH = ROWS_W // CH

TOK_W = T // NW
FCH = 32
NFCH = TOK_W // FCH


@functools.cache
def _build_sc_kernels():
    mesh = plsc.VectorSubcoreMesh(core_axis_name="c", subcore_axis_name="s")

    @functools.partial(
        pl.kernel,
        out_type=jax.ShapeDtypeStruct((T, D), jnp.float32),
        mesh=mesh,
        scratch_types=(
            [pltpu.VMEM((NFCH, 2 * FCH), jnp.int32)]
            + [pltpu.VMEM((2 * FCH, D), jnp.float32)]
            + [pltpu.VMEM((FCH, D), jnp.float32)]
            + [pltpu.SemaphoreType.DMA]
        ),
    )
    def sc_finalize(y_hbm, sint_hbm, out_hbm, *refs):
        idx_v = refs[0]
        g_v, o_v, sem = refs[1], refs[2], refs[3]
        wid = lax.axis_index("s") * NC + lax.axis_index("c")
        base = wid * TOK_W
        pltpu.sync_copy(sint_hbm.at[wid], idx_v)
        for c in range(NFCH):
            pltpu.async_copy(y_hbm.at[idx_v.at[c]], g_v, sem).wait()

            def _add(j, _):
                col = j * 16
                for r in range(FCH):
                    o_v[r, pl.ds(col, 16)] = (g_v[2 * r, pl.ds(col, 16)]
                                              + g_v[2 * r + 1, pl.ds(col, 16)])
                return 0

            lax.fori_loop(0, D // 16, _add, 0)
            pltpu.sync_copy(o_v, out_hbm.at[pl.ds(pl.multiple_of(base + c * FCH, 8), FCH)])

    return sc_finalize


def _tc_dispatch_body(nv_ref, hs_ref, tok_ref, x_ref):
    @pl.when(pl.program_id(0) < nv_ref[0])
    def _():
        tok = tok_ref[...]
        sel = (tok == lax.broadcasted_iota(jnp.int32, (BT, T), 1))
        s_mat = jnp.where(sel, 1.0, 0.0)
        x_ref[...] = lax.dot_general(
            s_mat, hs_ref[...], (((1,), (0,)), ((), ())),
            preferred_element_type=jnp.float32)


_tc_dispatch = pl.pallas_call(
    _tc_dispatch_body,
    grid_spec=pltpu.PrefetchScalarGridSpec(
        num_scalar_prefetch=1,
        grid=(NB,),
        in_specs=[
            pl.BlockSpec((T, D), lambda i, nv: (0, 0)),
            pl.BlockSpec((BT, 1), lambda i, nv: (i, 0)),
        ],
        out_specs=pl.BlockSpec((BT, D), lambda i, nv: (i, 0)),
    ),
    out_shape=jax.ShapeDtypeStruct((P, D), jnp.float32),
)


def _tc_moe_body(be_ref, nv_ref, x_ref, w1_ref, w2_ref, sw_ref, y_ref):
    @pl.when(pl.program_id(0) < nv_ref[0])
    def _():
        x = x_ref[...]
        w1b = w1_ref[0]
        g = lax.dot_general(x, w1b[:FF], (((1,), (1,)), ((), ())),
                            preferred_element_type=jnp.float32)
        u = lax.dot_general(x, w1b[FF:], (((1,), (1,)), ((), ())),
                            preferred_element_type=jnp.float32)
        h = g * lax.logistic(g) * u
        y = lax.dot_general(h, w2_ref[0], (((1,), (1,)), ((), ())),
                            preferred_element_type=jnp.float32)
        y_ref[...] = y * sw_ref[...]


_tc_moe = pl.pallas_call(
    _tc_moe_body,
    grid_spec=pltpu.PrefetchScalarGridSpec(
        num_scalar_prefetch=2,
        grid=(NB,),
        in_specs=[
            pl.BlockSpec((BT, D), lambda i, be, nv: (i, 0)),
            pl.BlockSpec((1, 2 * FF, D), lambda i, be, nv: (be[i], 0, 0)),
            pl.BlockSpec((1, D, FF), lambda i, be, nv: (be[i], 0, 0)),
            pl.BlockSpec((BT, 1), lambda i, be, nv: (i, 0)),
        ],
        out_specs=pl.BlockSpec((BT, D), lambda i, be, nv: (i, 0)),
    ),
    out_shape=jax.ShapeDtypeStruct((P, D), jnp.float32),
)


def kernel(hidden_states, w1, w2, topk_weights, topk_ids):
    flat_e = topk_ids.reshape(-1).astype(jnp.int32)
    iota = jnp.arange(N, dtype=jnp.int32)
    sorted_e, order, wt_sorted = lax.sort(
        (flat_e, iota, topk_weights.reshape(-1)), num_keys=1, is_stable=True)
    inv_order = jnp.argsort(order)
    counts = jnp.bincount(flat_e, length=E)
    raw_off = jnp.cumsum(counts) - counts
    pad_counts = ((counts + BT - 1) // BT) * BT
    pad_cum = jnp.cumsum(pad_counts)
    pad_off = pad_cum - pad_counts
    n_valid = (jnp.sum(pad_counts) // BT).astype(jnp.int32)
    be_raw = jnp.minimum(
        jnp.searchsorted(pad_cum, jnp.arange(NB) * BT, side="right"), E - 1
    ).astype(jnp.int32)
    block_expert = jnp.where(jnp.arange(NB) < n_valid, be_raw,
                             be_raw[jnp.maximum(n_valid - 1, 0)]).astype(jnp.int32)
    slot = (pad_off[sorted_e] + jnp.arange(N) - raw_off[sorted_e]).astype(jnp.int32)
    packed = jnp.stack([(order // K).astype(jnp.float32), wt_sorted])
    slot_info = jnp.zeros((2, P), jnp.float32).at[:, slot].set(packed)
    sorted_token = slot_info[0].astype(jnp.int32)
    sorted_wt = slot_info[1]
    sint = (pad_off[flat_e] + inv_order - raw_off[flat_e]).astype(
        jnp.int32).reshape(T, K)

    sc_finalize = _build_sc_kernels()
    nv = n_valid.reshape(1)
    x_sorted = _tc_dispatch(nv, hidden_states, sorted_token.reshape(P, 1))
    y_sorted = _tc_moe(block_expert, nv, x_sorted, w1, w2,
                       sorted_wt.reshape(P, 1))
    out = sc_finalize(y_sorted, sint.reshape(NW, NFCH, 2 * FCH))
    return out
